# baseline (speedup 1.0000x reference)
import jax
import jax.numpy as jnp
from jax import lax
from jax.experimental import pallas as pl
from jax.experimental.pallas import tpu as pltpu

_B = 1024
_S = 5
_H = 64
_N = _B * _S
_V = 1000000
_T = 256
_G = _N // _T
_KB = _B // _T


_NQ = 16
_JQ = _T // _NQ


def _issue(starts_ref, wt_ref, buf, sems, step, slot):
    for j in range(_T):
        s = pl.multiple_of(starts_ref[_T * step + j], 128)
        pltpu.make_async_copy(
            wt_ref.at[:, pl.ds(s, 128)], buf.at[slot, j],
            sems.at[slot, j // _JQ]
        ).start()


def _body(starts_ref, lanes_ref, it_ref, wt_ref, dum_ref, o_ref, buf, sems):
    m = pl.program_id(0)

    @pl.when(m == 0)
    def _prime():
        _issue(starts_ref, wt_ref, buf, sems, 0, 0)

    slot = m % 2
    for q in range(_NQ):
        pltpu.make_async_copy(
            dum_ref, buf.at[slot, pl.ds(q * _JQ, _JQ)], sems.at[slot, q]
        ).wait()

    @pl.when(m + 1 < _G)
    def _next():
        _issue(starts_ref, wt_ref, buf, sems, m + 1, (m + 1) % 2)

    blk = buf[slot]
    lc = lanes_ref[0][:, :, None]
    li = lax.broadcasted_iota(jnp.int32, (_T, 1, 128), 2)
    maskf = (li == lc).astype(jnp.float32)
    e = jnp.sum(blk * maskf, axis=2)
    x = jnp.dot(e, it_ref[...],
                preferred_element_type=jnp.float32)
    o_ref[0, :, :] = jax.nn.sigmoid(x)


def _make_fused():
    return pl.pallas_call(
        _body,
        grid_spec=pltpu.PrefetchScalarGridSpec(
            num_scalar_prefetch=1,
            grid=(_G,),
            in_specs=[
                pl.BlockSpec((1, _T, 1), lambda m, starts: (m, 0, 0)),
                pl.BlockSpec((_H, _B), lambda m, starts: (0, 0)),
                pl.BlockSpec(memory_space=pl.ANY),
                pl.BlockSpec(memory_space=pl.ANY),
            ],
            out_specs=pl.BlockSpec(
                (1, _T, _B), lambda m, starts: (m // _KB, m % _KB, 0)
            ),
            scratch_shapes=[
                pltpu.VMEM((2, _T, _H, 128), jnp.float32),
                pltpu.SemaphoreType.DMA((2, _NQ)),
            ],
        ),
        out_shape=jax.ShapeDtypeStruct((_S, _B, _B), jnp.float32),
    )


def kernel(inputs, idxs, W):
    idxf = idxs.astype(jnp.int32).T.reshape(-1)
    starts = (idxf >> 7) << 7
    lanes = (idxf & 127).reshape(_G, _T, 1)
    wt = W.T
    dummy = jnp.zeros((_JQ, _H, 128), jnp.float32)
    o5 = _make_fused()(starts, lanes, inputs.T, wt, dummy)
    return jnp.transpose(o5, (1, 2, 0))


# speedup vs baseline: 1.0850x; 1.0850x over previous
import jax
import jax.numpy as jnp
from jax import lax
from jax.experimental import pallas as pl
from jax.experimental.pallas import tpu as pltpu

_B = 1024
_S = 5
_H = 64
_N = _B * _S
_V = 1000000
_T = 256
_G = _N // _T
_KB = _B // _T


_NQ = 8
_JQ = _T // _NQ


def _issue(starts_ref, wt_ref, buf, sems, step, slot):
    for j in range(_T):
        s = pl.multiple_of(starts_ref[_T * step + j], 128)
        pltpu.make_async_copy(
            wt_ref.at[:, pl.ds(s, 128)], buf.at[slot, j],
            sems.at[slot, j // _JQ]
        ).start()


def _body(starts_ref, lanes_ref, it_ref, wt_ref, dum_ref, o_ref, buf, sems):
    m = pl.program_id(0)

    @pl.when(m == 0)
    def _prime():
        _issue(starts_ref, wt_ref, buf, sems, 0, 0)
        _issue(starts_ref, wt_ref, buf, sems, 1, 1)

    @pl.when(m + 2 < _G)
    def _next():
        _issue(starts_ref, wt_ref, buf, sems, m + 2, (m + 2) % 3)

    slot = m % 3
    for q in range(_NQ):
        pltpu.make_async_copy(
            dum_ref, buf.at[slot, pl.ds(q * _JQ, _JQ)], sems.at[slot, q]
        ).wait()

    blk = buf[slot]
    lc = lanes_ref[0][:, :, None]
    li = lax.broadcasted_iota(jnp.int32, (_T, 1, 128), 2)
    maskf = (li == lc).astype(jnp.float32)
    e = jnp.sum(blk * maskf, axis=2)
    x = jnp.dot(e, it_ref[...],
                preferred_element_type=jnp.float32)
    o_ref[0, :, :] = jax.nn.sigmoid(x)


def _make_fused():
    return pl.pallas_call(
        _body,
        grid_spec=pltpu.PrefetchScalarGridSpec(
            num_scalar_prefetch=1,
            grid=(_G,),
            in_specs=[
                pl.BlockSpec((1, _T, 1), lambda m, starts: (m, 0, 0)),
                pl.BlockSpec((_H, _B), lambda m, starts: (0, 0)),
                pl.BlockSpec(memory_space=pl.ANY),
                pl.BlockSpec(memory_space=pl.ANY),
            ],
            out_specs=pl.BlockSpec(
                (1, _T, _B), lambda m, starts: (m // _KB, m % _KB, 0)
            ),
            scratch_shapes=[
                pltpu.VMEM((3, _T, _H, 128), jnp.float32),
                pltpu.SemaphoreType.DMA((3, _NQ)),
            ],
        ),
        out_shape=jax.ShapeDtypeStruct((_S, _B, _B), jnp.float32),
    )


def kernel(inputs, idxs, W):
    idxf = idxs.astype(jnp.int32).T.reshape(-1)
    starts = (idxf >> 7) << 7
    lanes = (idxf & 127).reshape(_G, _T, 1)
    wt = W.T
    dummy = jnp.zeros((_JQ, _H, 128), jnp.float32)
    o5 = _make_fused()(starts, lanes, inputs.T, wt, dummy)
    return jnp.transpose(o5, (1, 2, 0))
